# baseline (device time: 34545 ns/iter reference)
import jax
import jax.numpy as jnp
from jax import lax
from jax.experimental import pallas as pl
from jax.experimental.pallas import tpu as pltpu


def kernel(partial, resid, gamma):
    _, m, d = partial.shape
    gamma2d = gamma.reshape(1, d)

    def body(partial_ref, resid_ref, gamma_ref, out_ref,
             send_buf, recv_buf, send_sem, recv_sem):
        my_x = lax.axis_index("x")
        my_y = lax.axis_index("y")
        peer = (my_x, 1 - my_y)

        barrier = pltpu.get_barrier_semaphore()
        pl.semaphore_signal(barrier, inc=1, device_id=peer,
                            device_id_type=pl.DeviceIdType.MESH)
        pl.semaphore_wait(barrier, 1)

        send_buf[...] = partial_ref[0].astype(jnp.bfloat16)
        rdma = pltpu.make_async_remote_copy(
            src_ref=send_buf,
            dst_ref=recv_buf,
            send_sem=send_sem,
            recv_sem=recv_sem,
            device_id=peer,
            device_id_type=pl.DeviceIdType.MESH,
        )
        rdma.start()
        rdma.wait()

        y = (partial_ref[0] + recv_buf[...].astype(jnp.float32)
             + resid_ref[...])
        rms = jnp.sqrt(jnp.mean(y * y, axis=-1, keepdims=True) + 1e-6)
        out_ref[...] = y / rms * gamma_ref[...]

    return pl.pallas_call(
        body,
        out_shape=jax.ShapeDtypeStruct((m, d), jnp.float32),
        in_specs=[pl.BlockSpec(memory_space=pltpu.VMEM)] * 3,
        out_specs=pl.BlockSpec(memory_space=pltpu.VMEM),
        scratch_shapes=[
            pltpu.VMEM((m, d), jnp.bfloat16),
            pltpu.VMEM((m, d), jnp.bfloat16),
            pltpu.SemaphoreType.DMA,
            pltpu.SemaphoreType.DMA,
        ],
        compiler_params=pltpu.CompilerParams(collective_id=0),
    )(partial, resid, gamma2d)


# device time: 26258 ns/iter; 1.3156x vs baseline; 1.3156x over previous
import jax
import jax.numpy as jnp
from jax import lax
from jax.experimental import pallas as pl
from jax.experimental.pallas import tpu as pltpu

C = 4


def kernel(partial, resid, gamma):
    _, m, d = partial.shape
    q = m // 4
    rc = q // C
    gamma2d = gamma.reshape(1, d)

    def body(partial_ref, resid_ref, gamma_ref, out_ref,
             rs_send, rs_recv, ag_stage, y_recv, x_recv, diag_recv,
             rs_ssem, rs_rsem, agy_ssem, agy_rsem,
             agx_ssem, agx_rsem, fwd_ssem, fwd_rsem):
        my_x = lax.axis_index("x")
        my_y = lax.axis_index("y")
        ypeer = (my_x, 1 - my_y)
        xpeer = (1 - my_x, my_y)

        my_blk = 2 * q * my_x + q * my_y
        ypeer_blk = 2 * q * my_x + q * (1 - my_y)
        xpeer_blk = 2 * q * (1 - my_x) + q * my_y
        diag_blk = 2 * q * (1 - my_x) + q * (1 - my_y)

        barrier = pltpu.get_barrier_semaphore()
        for nbr in (ypeer, xpeer):
            pl.semaphore_signal(barrier, inc=1, device_id=nbr,
                                device_id_type=pl.DeviceIdType.MESH)
        pl.semaphore_wait(barrier, 2)

        rs_send[...] = partial_ref[0, pl.ds(ypeer_blk, q), :].astype(
            jnp.bfloat16)
        rs = []
        for c in range(C):
            sl = slice(c * rc, (c + 1) * rc)
            r = pltpu.make_async_remote_copy(
                src_ref=rs_send.at[sl], dst_ref=rs_recv.at[sl],
                send_sem=rs_ssem.at[c], recv_sem=rs_rsem.at[c],
                device_id=ypeer, device_id_type=pl.DeviceIdType.MESH)
            r.start()
            rs.append(r)

        agy, agx = [], []
        for c in range(C):
            sl = slice(c * rc, (c + 1) * rc)
            rs[c].wait_recv()
            rows = pl.ds(my_blk + c * rc, rc)
            yv = (partial_ref[0, rows, :]
                  + rs_recv[sl, :].astype(jnp.float32)
                  + resid_ref[rows, :])
            rms = jnp.sqrt(jnp.mean(yv * yv, axis=-1, keepdims=True) + 1e-6)
            outv = yv / rms * gamma_ref[...]
            out_ref[rows, :] = outv
            ag_stage[sl, :] = outv.astype(jnp.bfloat16)
            for sems, buf, peer, lst in (
                ((agy_ssem, agy_rsem), y_recv, ypeer, agy),
                ((agx_ssem, agx_rsem), x_recv, xpeer, agx),
            ):
                rr = pltpu.make_async_remote_copy(
                    src_ref=ag_stage.at[sl], dst_ref=buf.at[sl],
                    send_sem=sems[0].at[c], recv_sem=sems[1].at[c],
                    device_id=peer, device_id_type=pl.DeviceIdType.MESH)
                rr.start()
                lst.append(rr)

        fwd = []
        for c in range(C):
            sl = slice(c * rc, (c + 1) * rc)
            agy[c].wait_recv()
            f = pltpu.make_async_remote_copy(
                src_ref=y_recv.at[sl], dst_ref=diag_recv.at[sl],
                send_sem=fwd_ssem.at[c], recv_sem=fwd_rsem.at[c],
                device_id=xpeer, device_id_type=pl.DeviceIdType.MESH)
            f.start()
            fwd.append(f)
            out_ref[pl.ds(ypeer_blk + c * rc, rc), :] = (
                y_recv[sl, :].astype(jnp.float32))

        for c in range(C):
            sl = slice(c * rc, (c + 1) * rc)
            agx[c].wait_recv()
            out_ref[pl.ds(xpeer_blk + c * rc, rc), :] = (
                x_recv[sl, :].astype(jnp.float32))
        for c in range(C):
            sl = slice(c * rc, (c + 1) * rc)
            fwd[c].wait_recv()
            out_ref[pl.ds(diag_blk + c * rc, rc), :] = (
                diag_recv[sl, :].astype(jnp.float32))

        for c in range(C):
            rs[c].wait_send()
            agy[c].wait_send()
            agx[c].wait_send()
            fwd[c].wait_send()

    return pl.pallas_call(
        body,
        out_shape=jax.ShapeDtypeStruct((m, d), jnp.float32),
        in_specs=[pl.BlockSpec(memory_space=pltpu.VMEM)] * 3,
        out_specs=pl.BlockSpec(memory_space=pltpu.VMEM),
        scratch_shapes=[
            pltpu.VMEM((q, d), jnp.bfloat16),
            pltpu.VMEM((q, d), jnp.bfloat16),
            pltpu.VMEM((q, d), jnp.bfloat16),
            pltpu.VMEM((q, d), jnp.bfloat16),
            pltpu.VMEM((q, d), jnp.bfloat16),
            pltpu.VMEM((q, d), jnp.bfloat16),
            pltpu.SemaphoreType.DMA((C,)),
            pltpu.SemaphoreType.DMA((C,)),
            pltpu.SemaphoreType.DMA((C,)),
            pltpu.SemaphoreType.DMA((C,)),
            pltpu.SemaphoreType.DMA((C,)),
            pltpu.SemaphoreType.DMA((C,)),
            pltpu.SemaphoreType.DMA((C,)),
            pltpu.SemaphoreType.DMA((C,)),
        ],
        compiler_params=pltpu.CompilerParams(collective_id=0),
    )(partial, resid, gamma2d)
